# Initial kernel scaffold; baseline (speedup 1.0000x reference)
import jax
import jax.numpy as jnp
from jax.experimental import pallas as pl


def kernel(x, edge_index, edge_type, W_rel_in, W_self_in, W_rel_hid, W_self_hid, fcn_W, fcn_b, out_W, out_b):
    raise NotImplementedError("write your pallas kernel here")


# speedup vs baseline: 2.4314x; 2.4314x over previous
import functools

import jax
import jax.numpy as jnp
from jax import lax
from jax.experimental import pallas as pl
from jax.experimental.pallas import tpu as pltpu
from jax.experimental.pallas import tpu_sc as plsc

N = 10000
E = 160000
R = 8
H = 256
HALF = 128
BG = 20
NNODE = 500
RR = R + 1

TN = 1000
NT = N // TN

NSUB = 16
EPT = E // NSUB
OC = 2000
NOUT = EPT // OC
G = 80
NIN = OC // G
ZR = 200
RPT = 1000


def _mm_body(h_ref, w_ref, t0_ref, t1_ref):
    acc = jnp.dot(h_ref[...], w_ref[0], preferred_element_type=jnp.float32)
    t0_ref[0] = acc[:, :HALF]
    t1_ref[0] = acc[:, HALF:]


def _rgcn_tables(h, w_all):
    d = h.shape[1]
    return pl.pallas_call(
        _mm_body,
        grid=(NT, RR),
        in_specs=[
            pl.BlockSpec((TN, d), lambda t, r: (t, 0)),
            pl.BlockSpec((1, d, H), lambda t, r: (r, 0, 0)),
        ],
        out_specs=[
            pl.BlockSpec((1, TN, HALF), lambda t, r: (r, t, 0)),
            pl.BlockSpec((1, TN, HALF), lambda t, r: (r, t, 0)),
        ],
        out_shape=[
            jax.ShapeDtypeStruct((RR, N, HALF), jnp.float32),
            jax.ShapeDtypeStruct((RR, N, HALF), jnp.float32),
        ],
    )(h, w_all)


_sc_mesh = plsc.VectorSubcoreMesh(core_axis_name="c", subcore_axis_name="s")


@functools.partial(
    pl.kernel,
    mesh=_sc_mesh,
    out_type=jax.ShapeDtypeStruct((2, N, HALF), jnp.float32),
    scratch_types=[
        pltpu.VMEM((OC,), jnp.int32),
        pltpu.VMEM((OC,), jnp.int32),
        pltpu.VMEM((OC,), jnp.int32),
        pltpu.VMEM((NIN, G), jnp.int32),
        pltpu.VMEM((NIN, G), jnp.int32),
        pltpu.VMEM((G, HALF), jnp.float32),
        pltpu.VMEM((ZR, HALF), jnp.float32),
        pltpu.VMEM_SHARED((N, HALF), jnp.float32),
        pltpu.SemaphoreType.DMA,
    ],
)
def _sc_agg(t0_hbm, t1_hbm, src_hbm, dst_hbm, et_hbm, out_hbm,
            src_v, ety_v, dst_v, gidx_v, sidx_v, rows_v, zbuf, acc_sh, sem):
    cid = lax.axis_index("c")
    sid = lax.axis_index("s")

    def _zero(i, _):
        r = i // 8
        k = i - r * 8
        zbuf[r, pl.ds(k * 16, 16)] = jnp.zeros((16,), jnp.float32)
        return 0

    lax.fori_loop(0, ZR * 8, _zero, 0)

    @pl.when(sid < N // RPT)
    def _():
        for k in range(RPT // ZR):
            pltpu.sync_copy(zbuf, acc_sh.at[pl.ds(sid * RPT + k * ZR, ZR)])

    plsc.subcore_barrier()

    for o in range(NOUT):
        eb = sid * EPT + o * OC
        pltpu.sync_copy(src_hbm.at[pl.ds(eb, OC)], src_v)
        pltpu.sync_copy(dst_hbm.at[pl.ds(eb, OC)], dst_v)
        pltpu.sync_copy(et_hbm.at[pl.ds(eb, OC)], ety_v)

        def _mkidx(t, _):
            g = t // 5
            k = t - g * 5
            off = t * 16
            s16 = src_v[pl.ds(off, 16)]
            e16 = ety_v[pl.ds(off, 16)]
            gidx_v[g, pl.ds(k * 16, 16)] = e16 * N + s16
            sidx_v[g, pl.ds(k * 16, 16)] = dst_v[pl.ds(off, 16)]
            return 0

        lax.fori_loop(0, OC // 16, _mkidx, 0)

        def _edges(j, _):
            @pl.when(cid == 0)
            def _():
                pltpu.async_copy(t0_hbm.at[gidx_v.at[j]], rows_v, sem).wait()

            @pl.when(cid == 1)
            def _():
                pltpu.async_copy(t1_hbm.at[gidx_v.at[j]], rows_v, sem).wait()

            pltpu.sync_copy(rows_v, acc_sh.at[sidx_v.at[j]], add=True)
            return 0

        lax.fori_loop(0, NIN, _edges, 0)

    plsc.subcore_barrier()

    @pl.when(sid < N // RPT)
    def _():
        pltpu.sync_copy(acc_sh.at[pl.ds(sid * RPT, RPT)],
                        out_hbm.at[cid, pl.ds(sid * RPT, RPT)])


def _relu_body(agg_ref, t0_ref, t1_ref, h_ref):
    h_ref[:, :HALF] = jnp.maximum(agg_ref[0] + t0_ref[0], 0.0)
    h_ref[:, HALF:] = jnp.maximum(agg_ref[1] + t1_ref[0], 0.0)


def _relu_combine(agg, t0, t1):
    return pl.pallas_call(
        _relu_body,
        grid=(NT,),
        in_specs=[
            pl.BlockSpec((2, TN, HALF), lambda t: (0, t, 0)),
            pl.BlockSpec((1, TN, HALF), lambda t: (R, t, 0)),
            pl.BlockSpec((1, TN, HALF), lambda t: (R, t, 0)),
        ],
        out_specs=pl.BlockSpec((TN, H), lambda t: (t, 0)),
        out_shape=jax.ShapeDtypeStruct((N, H), jnp.float32),
    )(agg, t0, t1)


def _head_body(h_ref, w_ref, b_ref, s_ref, ow_ref, ob_ref, o_ref):
    h = h_ref[...]
    h = jnp.maximum(
        jnp.dot(h, w_ref[0], preferred_element_type=jnp.float32) + b_ref[0], 0.0)
    h = jnp.maximum(
        jnp.dot(h, w_ref[1], preferred_element_type=jnp.float32) + b_ref[1], 0.0)
    g = jnp.dot(s_ref[...], h, preferred_element_type=jnp.float32)
    o_ref[...] = jnp.dot(g, ow_ref[...], preferred_element_type=jnp.float32) + ob_ref[...]


def _head(h, fcn_w, fcn_b, seg, out_w, out_b):
    return pl.pallas_call(
        _head_body,
        out_shape=jax.ShapeDtypeStruct((BG, 2), jnp.float32),
    )(h, fcn_w, fcn_b, seg, out_w, out_b.reshape(1, 2))


def kernel(x, edge_index, edge_type, W_rel_in, W_self_in, W_rel_hid,
           W_self_hid, fcn_W, fcn_b, out_W, out_b):
    h = x
    for i in range(4):
        if i == 0:
            w_all = jnp.concatenate([W_rel_in, W_self_in[None]], axis=0)
        else:
            w_all = jnp.concatenate(
                [W_rel_hid[i - 1], W_self_hid[i - 1][None]], axis=0)
        t0, t1 = _rgcn_tables(h, w_all)
        agg = _sc_agg(t0.reshape(RR * N, HALF), t1.reshape(RR * N, HALF),
                      edge_index[0], edge_index[1], edge_type)
        h = _relu_combine(agg, t0, t1)
    seg = ((jnp.arange(N, dtype=jnp.int32)[None, :] // NNODE)
           == jnp.arange(BG, dtype=jnp.int32)[:, None]).astype(jnp.float32) / NNODE
    return _head(h, fcn_W, fcn_b, seg, out_W, out_b)
